# baseline (speedup 1.0000x reference)
import functools

import jax
import jax.numpy as jnp
from jax import lax
from jax.experimental import pallas as pl
from jax.experimental.pallas import tpu as pltpu
from jax.experimental.pallas import tpu_sc as plsc

ROWS, COLS = 128, 32768


_NC, _NS, _L = 2, 16, 16
_NW = _NC * _NS
_RPW = ROWS // _NW
_ACC = 8
_NSLICE = COLS // _L

_sc_mesh = plsc.VectorSubcoreMesh(core_axis_name="c", subcore_axis_name="s")


def _row_argmax(row_buf, b):
    neg = jnp.full((_L,), -jnp.inf, dtype=jnp.float32)
    zero = jnp.zeros((_L,), dtype=jnp.int32)

    def body(i, carry):
        vmaxs, vidxs = carry
        new_m, new_i = [], []
        for k in range(_ACC):
            sl = i * _ACC + k
            v = row_buf[b, 0, pl.ds(sl * _L, _L)]
            m = v > vmaxs[k]
            new_m.append(jnp.maximum(vmaxs[k], v))
            new_i.append(jnp.where(m, sl, vidxs[k]))
        return tuple(new_m), tuple(new_i)

    vmaxs, vidxs = lax.fori_loop(
        0, _NSLICE // _ACC, body, ((neg,) * _ACC, (zero,) * _ACC)
    )
    vm, vi = vmaxs[0], vidxs[0]
    for k in range(1, _ACC):
        m = vmaxs[k] > vm
        vi = jnp.where(m, vidxs[k], vi)
        vm = jnp.maximum(vm, vmaxs[k])
    gidx_vec = vi * _L + lax.iota(jnp.int32, _L)
    best_v = vm[0]
    best_i = gidx_vec[0]
    for j in range(1, _L):
        take = vm[j] > best_v
        best_i = jnp.where(take, gidx_vec[j], best_i)
        best_v = jnp.where(take, vm[j], best_v)
    return jnp.full((_L,), best_i, dtype=jnp.int32)


@functools.partial(
    pl.kernel,
    out_type=jax.ShapeDtypeStruct((_NW, _RPW, _L), jnp.int32),
    mesh=_sc_mesh,
    scratch_types=[
        pltpu.VMEM((2, 1, COLS), jnp.float32),
        pltpu.VMEM((_RPW, _L), jnp.int32),
        pltpu.SemaphoreType.DMA,
    ],
)
def _sc_argmax(x_hbm, out_hbm, row_buf, out_buf, sem):
    wid = lax.axis_index("s") * _NC + lax.axis_index("c")
    base = wid * _RPW
    copies = [pltpu.async_copy(x_hbm.at[pl.ds(base, 1)], row_buf.at[0], sem)]
    for r in range(_RPW):
        if r + 1 < _RPW:
            copies.append(
                pltpu.async_copy(
                    x_hbm.at[pl.ds(base + r + 1, 1)],
                    row_buf.at[(r + 1) % 2],
                    sem,
                )
            )
        copies[r].wait()
        out_buf[r] = _row_argmax(row_buf, r % 2)
    pltpu.sync_copy(out_buf, out_hbm.at[wid])


_RBLK = 64


def _copy_body(x_ref, y_ref):
    y_ref[...] = x_ref[...]


def _tc_copy(x):
    return pl.pallas_call(
        _copy_body,
        grid=(ROWS // _RBLK,),
        in_specs=[pl.BlockSpec((_RBLK, COLS), lambda k: (k, 0))],
        out_specs=pl.BlockSpec((_RBLK, COLS), lambda k: (k, 0)),
        out_shape=jax.ShapeDtypeStruct((ROWS, COLS), jnp.float32),
    )(x)


def kernel(inputs):
    idx = _sc_argmax(inputs)
    y = _tc_copy(inputs)
    r0 = idx[0, 0, 0] // jnp.int32(COLS)
    row = lax.dynamic_slice(inputs, (r0, jnp.int32(0)), (1, COLS))
    return lax.dynamic_update_slice(y, row, (r0, jnp.int32(0)))


# speedup vs baseline: 1.0576x; 1.0576x over previous
import functools

import jax
import jax.numpy as jnp
from jax import lax
from jax.experimental import pallas as pl
from jax.experimental.pallas import tpu as pltpu
from jax.experimental.pallas import tpu_sc as plsc

ROWS, COLS = 128, 32768


_NC, _NS, _L = 2, 16, 16
_NW = _NC * _NS
_RPW = ROWS // _NW
_ACC = 8
_NSLICE = COLS // _L

_sc_mesh = plsc.VectorSubcoreMesh(core_axis_name="c", subcore_axis_name="s")


def _row_argmax(row_buf, b):
    neg = jnp.full((_L,), -jnp.inf, dtype=jnp.float32)
    zero = jnp.zeros((_L,), dtype=jnp.int32)

    def body(i, carry):
        vmaxs, vidxs = carry
        i_splat = jnp.full((_L,), 0, jnp.int32) + i
        new_m, new_i = [], []
        for k in range(_ACC):
            v = row_buf[b, 0, pl.ds((i * _ACC + k) * _L, _L)]
            m = v > vmaxs[k]
            new_m.append(jnp.maximum(vmaxs[k], v))
            new_i.append(jnp.where(m, i_splat, vidxs[k]))
        return tuple(new_m), tuple(new_i)

    vmaxs, vidxs = lax.fori_loop(
        0, _NSLICE // _ACC, body, ((neg,) * _ACC, (zero,) * _ACC)
    )
    lane = lax.iota(jnp.int32, _L)
    vm, vi = vmaxs[0], vidxs[0] * (_ACC * _L) + lane
    for k in range(1, _ACC):
        gidx_k = vidxs[k] * (_ACC * _L) + (k * _L) + lane
        m = vmaxs[k] > vm
        vi = jnp.where(m, gidx_k, vi)
        vm = jnp.maximum(vm, vmaxs[k])
    best_v = vm[0]
    best_i = vi[0]
    for j in range(1, _L):
        take = vm[j] > best_v
        best_i = jnp.where(take, vi[j], best_i)
        best_v = jnp.where(take, vm[j], best_v)
    return jnp.full((_L,), best_i, dtype=jnp.int32)


@functools.partial(
    pl.kernel,
    out_type=jax.ShapeDtypeStruct((_NW, _RPW, _L), jnp.int32),
    mesh=_sc_mesh,
    scratch_types=[
        pltpu.VMEM((2, 1, COLS), jnp.float32),
        pltpu.VMEM((_RPW, _L), jnp.int32),
        pltpu.SemaphoreType.DMA,
    ],
)
def _sc_argmax(x_hbm, out_hbm, row_buf, out_buf, sem):
    wid = lax.axis_index("s") * _NC + lax.axis_index("c")
    base = wid * _RPW
    copies = [pltpu.async_copy(x_hbm.at[pl.ds(base, 1)], row_buf.at[0], sem)]
    for r in range(_RPW):
        if r + 1 < _RPW:
            copies.append(
                pltpu.async_copy(
                    x_hbm.at[pl.ds(base + r + 1, 1)],
                    row_buf.at[(r + 1) % 2],
                    sem,
                )
            )
        copies[r].wait()
        out_buf[r] = _row_argmax(row_buf, r % 2)
    pltpu.sync_copy(out_buf, out_hbm.at[wid])


_RBLK = 64


def _copy_body(x_ref, y_ref):
    y_ref[...] = x_ref[...]


def _tc_copy(x):
    return pl.pallas_call(
        _copy_body,
        grid=(ROWS // _RBLK,),
        in_specs=[pl.BlockSpec((_RBLK, COLS), lambda k: (k, 0))],
        out_specs=pl.BlockSpec((_RBLK, COLS), lambda k: (k, 0)),
        out_shape=jax.ShapeDtypeStruct((ROWS, COLS), jnp.float32),
    )(x)


def kernel(inputs):
    idx = _sc_argmax(inputs)
    y = _tc_copy(inputs)
    r0 = idx[0, 0, 0] // jnp.int32(COLS)
    patch = lax.dynamic_slice(inputs, (r0, r0), (1, 1))
    return lax.dynamic_update_slice(y, patch, (r0, r0))


# speedup vs baseline: 3.5213x; 3.3296x over previous
import jax
import jax.numpy as jnp
from jax import lax
from jax.experimental import pallas as pl

ROWS, COLS = 128, 32768

_RBLK = 64
_W = 128
_NCHUNK = COLS // _W


def _body(x_ref, y_ref, amax_ref):
    blk = x_ref[...]
    y_ref[...] = blk
    acc = blk[:, :_W]
    idx = jnp.zeros((_RBLK, _W), dtype=jnp.int32)
    for c in range(1, _NCHUNK):
        v = blk[:, c * _W:(c + 1) * _W]
        m = v > acc
        acc = jnp.where(m, v, acc)
        idx = jnp.where(m, c, idx)
    bmax = jnp.max(acc, axis=1, keepdims=True)
    gidx = idx * _W + lax.broadcasted_iota(jnp.int32, (_RBLK, _W), 1)
    cand = jnp.where(acc == bmax, gidx, COLS)
    amax_ref[...] = jnp.min(cand, axis=1, keepdims=True)


def kernel(inputs):
    y, _idx = pl.pallas_call(
        _body,
        grid=(ROWS // _RBLK,),
        in_specs=[pl.BlockSpec((_RBLK, COLS), lambda k: (k, 0))],
        out_specs=[
            pl.BlockSpec((_RBLK, COLS), lambda k: (k, 0)),
            pl.BlockSpec((_RBLK, 1), lambda k: (k, 0)),
        ],
        out_shape=[
            jax.ShapeDtypeStruct((ROWS, COLS), jnp.float32),
            jax.ShapeDtypeStruct((ROWS, 1), jnp.int32),
        ],
    )(inputs)
    return y
